# baseline (speedup 1.0000x reference)
import jax
import jax.numpy as jnp
from jax import lax
from jax.experimental import pallas as pl
from jax.experimental.pallas import tpu as pltpu
from jax.experimental.pallas import tpu_sc as plsc

B, S, V, H, P, T = 4096, 200, 100000, 128, 512, 2

NC, NS, L = 2, 16, 16
NW = NC * NS
ROWS_PER_W = B // NW
NH = H // L
TOK_W = ROWS_PER_W * S

C0, C1 = 104, 96
NCHUNK = 2 * ROWS_PER_W
NBUF = 4
INNER = 4
UNROLL = 10


def _lane_sum(x):
    lanes = jnp.arange(L, dtype=jnp.int32)
    dnums = lax.GatherDimensionNumbers(
        offset_dims=(), collapsed_slice_dims=(0,), start_index_map=(0,))
    for d in (8, 4, 2, 1):
        x = x + lax.gather(x, (lanes ^ d)[:, None], dnums, slice_sizes=(1,),
                           mode=lax.GatherScatterMode.PROMISE_IN_BOUNDS)
    return x


def _perm(x, d):
    lanes = jnp.arange(L, dtype=jnp.int32)
    dnums = lax.GatherDimensionNumbers(
        offset_dims=(), collapsed_slice_dims=(0,), start_index_map=(0,))
    return lax.gather(x, (lanes ^ d)[:, None], dnums, slice_sizes=(1,),
                      mode=lax.GatherScatterMode.PROMISE_IN_BOUNDS)


def _bcast(x, lane):
    dnums = lax.GatherDimensionNumbers(
        offset_dims=(), collapsed_slice_dims=(0,), start_index_map=(0,))
    return lax.gather(x, jnp.full((L, 1), lane, jnp.int32), dnums,
                      slice_sizes=(1,), mode=lax.GatherScatterMode.PROMISE_IN_BOUNDS)


def _rsqrt(x):
    i = lax.bitcast_convert_type(x, jnp.int32)
    i = jnp.int32(0x5F3759DF) - (i >> 1)
    y = lax.bitcast_convert_type(i, jnp.float32)
    for _ in range(1):
        y = y * (1.5 - 0.5 * x * y * y)
    return y


def _sc_body(ids_hbm, tt_hbm, word_hbm, pos_hbm, typ_hbm, g_hbm, b_hbm,
             out_hbm, buf0, buf1, buf2, buf3, ids_cb, tt_cb, pt_v, typ_v,
             g_v, b_v, g_sems, o_sems, i_sems, t_sems):
    wid = lax.axis_index("s") * NC + lax.axis_index("c")
    b0 = wid * ROWS_PER_W
    tok0 = b0 * S
    bufs = (buf0, buf1, buf2, buf3)

    pltpu.sync_copy(pos_hbm.at[pl.ds(0, S)], pt_v.at[pl.ds(0, S)])
    pltpu.sync_copy(typ_hbm, typ_v)
    pltpu.sync_copy(g_hbm, g_v)
    pltpu.sync_copy(b_hbm, b_v)

    @plsc.parallel_loop(0, S, unroll=4)
    def build_pt(s):
        for j in range(NH):
            sl = pl.ds(j * L, L)
            p = pt_v[s, sl]
            pt_v[S + s, sl] = p + typ_v[1, sl]
            pt_v[s, sl] = p + typ_v[0, sl]

    def chunk_geom(m):
        off = 0 if m % 2 == 0 else C0
        ln = C0 if m % 2 == 0 else C1
        return off, ln

    def tok_start(c_outer, m):
        off, _ = chunk_geom(m)
        row = c_outer * (INNER // 2) + m // 2
        return row * S + off

    def issue_ids(c_outer, m, k):
        _, ln = chunk_geom(m)
        p = tok_start(c_outer, m)
        pltpu.async_copy(ids_hbm.at[pl.ds(tok0 + p, ln)],
                         ids_cb.at[k, pl.ds(0, ln)], i_sems.at[k])
        pltpu.async_copy(tt_hbm.at[pl.ds(tok0 + p, ln)],
                         tt_cb.at[k, pl.ds(0, ln)], t_sems.at[k])

    def wait_ids(m, k):
        _, ln = chunk_geom(m)
        pltpu.make_async_copy(ids_hbm.at[pl.ds(tok0, ln)],
                              ids_cb.at[k, pl.ds(0, ln)], i_sems.at[k]).wait()

    def issue_gather(m, k):
        _, ln = chunk_geom(m)
        pltpu.async_copy(word_hbm.at[ids_cb.at[k, pl.ds(0, ln)]],
                         bufs[k].at[pl.ds(0, ln)], g_sems.at[k])

    for m in range(NBUF):
        issue_ids(0, m, m)
    for m in range(2):
        wait_ids(m, m)
        issue_gather(m, m)

    def outer_body(t, carry):
        for m in range(INNER):
            off, ln = chunk_geom(m)
            k = m % NBUF
            k2 = (m + 2) % NBUF
            _, ln2 = chunk_geom(m + 2)
            c = t * INNER + m
            row = t * (INNER // 2) + m // 2
            b = b0 + row
            p0 = row * S + off

            pltpu.make_async_copy(
                word_hbm.at[ids_cb.at[k, pl.ds(0, ln)]],
                bufs[k].at[pl.ds(0, ln)], g_sems.at[k]).wait()
            pltpu.make_async_copy(
                tt_hbm.at[pl.ds(tok0, ln)],
                tt_cb.at[k, pl.ds(0, ln)], t_sems.at[k]).wait()

            @pl.when(c >= 2)
            def _():
                pltpu.make_async_copy(
                    bufs[k2].at[pl.ds(0, ln2)],
                    out_hbm.at[b, pl.ds(0, ln2)], o_sems.at[k2]).wait()

            @pl.when(c + 2 < NCHUNK)
            def _():
                wait_ids(m + 2, k2)
                issue_gather(m + 2, k2)

            buf = bufs[k]

            g_regs = [g_v[pl.ds(j * L, L)] for j in range(NH)]
            b_regs = [b_v[pl.ds(j * L, L)] for j in range(NH)]

            @plsc.parallel_loop(0, ln, unroll=UNROLL)
            def tok_body(s):
                tt = tt_cb[k, pl.ds(s, L)][0]
                ptrow = tt * S + (off + s)
                xs = []
                qs = []
                for j in range(NH):
                    sl = pl.ds(j * L, L)
                    x = buf[s, sl] + pt_v[ptrow, sl]
                    xs.append(x)
                    qs.append(x * x)
                def tree(vs):
                    while len(vs) > 1:
                        vs = [a + b for a, b in zip(vs[::2], vs[1::2])]
                    return vs[0]
                acc_s = tree(xs)
                acc_q = tree(qs)
                acc_s = acc_s + _perm(acc_s, 8)
                acc_q = acc_q + _perm(acc_q, 8)
                z = jnp.where(jnp.arange(L) < 8, acc_s, acc_q)
                for d in (4, 2, 1):
                    z = z + _perm(z, d)
                mean = _bcast(z, 0) * (1.0 / H)
                ex2 = _bcast(z, 8) * (1.0 / H)
                inv = _rsqrt(ex2 - mean * mean + 1e-12)
                for j in range(NH):
                    sl = pl.ds(j * L, L)
                    buf[s, sl] = (xs[j] - mean) * inv * g_regs[j] + b_regs[j]

            pltpu.async_copy(
                buf.at[pl.ds(0, ln)],
                out_hbm.at[b, pl.ds(off, ln)], o_sems.at[k])

            @pl.when(c + NBUF < NCHUNK)
            def _():
                row4 = (c + NBUF) // 2
                off4, ln4 = chunk_geom(m)
                p4 = row4 * S + off4
                pltpu.async_copy(ids_hbm.at[pl.ds(tok0 + p4, ln4)],
                                 ids_cb.at[k, pl.ds(0, ln4)], i_sems.at[k])
                pltpu.async_copy(tt_hbm.at[pl.ds(tok0 + p4, ln4)],
                                 tt_cb.at[k, pl.ds(0, ln4)], t_sems.at[k])
        return carry

    lax.fori_loop(0, NCHUNK // INNER, outer_body, 0)

    for m in (INNER - 2, INNER - 1):
        off, ln = chunk_geom(m)
        k = m % NBUF
        pltpu.make_async_copy(
            bufs[k].at[pl.ds(0, ln)],
            out_hbm.at[b0, pl.ds(off, ln)], o_sems.at[k]).wait()


def kernel(input_ids, token_type_ids, word_embeddings, position_embeddings,
           token_type_embeddings, ln_gamma, ln_beta):
    mesh = plsc.VectorSubcoreMesh(core_axis_name="c", subcore_axis_name="s")
    f = pl.kernel(
        _sc_body,
        out_type=jax.ShapeDtypeStruct((B, S, H), jnp.float32),
        mesh=mesh,
        scratch_types=[
            pltpu.VMEM((C0, H), jnp.float32),
            pltpu.VMEM((C0, H), jnp.float32),
            pltpu.VMEM((C0, H), jnp.float32),
            pltpu.VMEM((C0, H), jnp.float32),
            pltpu.VMEM((NBUF, C0 + 8), jnp.int32),
            pltpu.VMEM((NBUF, C0 + L), jnp.int32),
            pltpu.VMEM((2 * S, H), jnp.float32),
            pltpu.VMEM((T, H), jnp.float32),
            pltpu.VMEM((H,), jnp.float32),
            pltpu.VMEM((H,), jnp.float32),
            pltpu.SemaphoreType.DMA((NBUF,)),
            pltpu.SemaphoreType.DMA((NBUF,)),
            pltpu.SemaphoreType.DMA((NBUF,)),
            pltpu.SemaphoreType.DMA((NBUF,)),
        ],
    )
    return f(input_ids.astype(jnp.int32).reshape(-1),
             token_type_ids.astype(jnp.int32).reshape(-1),
             word_embeddings, position_embeddings, token_type_embeddings,
             ln_gamma, ln_beta)


# speedup vs baseline: 1.1516x; 1.1516x over previous
import jax
import jax.numpy as jnp
from jax import lax
from jax.experimental import pallas as pl
from jax.experimental.pallas import tpu as pltpu
from jax.experimental.pallas import tpu_sc as plsc

B, S, V, H, P, T = 4096, 200, 100000, 128, 512, 2

NC, NS, L = 2, 16, 16
NW = NC * NS
ROWS_PER_W = B // NW
NH = H // L
TOK_W = ROWS_PER_W * S

C0, C1 = 104, 96
NCHUNK = 2 * ROWS_PER_W
NBUF = 4
INNER = 4
UNROLL = 8


def _lane_sum(x):
    lanes = jnp.arange(L, dtype=jnp.int32)
    dnums = lax.GatherDimensionNumbers(
        offset_dims=(), collapsed_slice_dims=(0,), start_index_map=(0,))
    for d in (8, 4, 2, 1):
        x = x + lax.gather(x, (lanes ^ d)[:, None], dnums, slice_sizes=(1,),
                           mode=lax.GatherScatterMode.PROMISE_IN_BOUNDS)
    return x


def _perm(x, d):
    lanes = jnp.arange(L, dtype=jnp.int32)
    dnums = lax.GatherDimensionNumbers(
        offset_dims=(), collapsed_slice_dims=(0,), start_index_map=(0,))
    return lax.gather(x, (lanes ^ d)[:, None], dnums, slice_sizes=(1,),
                      mode=lax.GatherScatterMode.PROMISE_IN_BOUNDS)


def _bcast(x, lane):
    dnums = lax.GatherDimensionNumbers(
        offset_dims=(), collapsed_slice_dims=(0,), start_index_map=(0,))
    return lax.gather(x, jnp.full((L, 1), lane, jnp.int32), dnums,
                      slice_sizes=(1,), mode=lax.GatherScatterMode.PROMISE_IN_BOUNDS)


def _rsqrt(x):
    i = lax.bitcast_convert_type(x, jnp.int32)
    i = jnp.int32(0x5F3759DF) - (i >> 1)
    y = lax.bitcast_convert_type(i, jnp.float32)
    for _ in range(1):
        y = y * (1.5 - 0.5 * x * y * y)
    return y


def _sc_body(ids_hbm, tt_hbm, word_hbm, pos_hbm, typ_hbm, g_hbm, b_hbm,
             out_hbm, buf0, buf1, buf2, buf3, ids_cb, tt_cb, pt_v, typ_v,
             g_v, b_v, g_sems, o_sems, i_sems, t_sems):
    wid = lax.axis_index("s") * NC + lax.axis_index("c")
    b0 = wid * ROWS_PER_W
    tok0 = b0 * S
    bufs = (buf0, buf1, buf2, buf3)

    pltpu.sync_copy(pos_hbm.at[pl.ds(0, S)], pt_v.at[pl.ds(0, S)])
    pltpu.sync_copy(typ_hbm, typ_v)
    pltpu.sync_copy(g_hbm, g_v)
    pltpu.sync_copy(b_hbm, b_v)

    @plsc.parallel_loop(0, S, unroll=4)
    def build_pt(s):
        for j in range(NH):
            sl = pl.ds(j * L, L)
            p = pt_v[s, sl]
            pt_v[S + s, sl] = p + typ_v[1, sl]
            pt_v[s, sl] = p + typ_v[0, sl]

    def chunk_geom(m):
        off = 0 if m % 2 == 0 else C0
        ln = C0 if m % 2 == 0 else C1
        return off, ln

    def tok_start(c_outer, m):
        off, _ = chunk_geom(m)
        row = c_outer * (INNER // 2) + m // 2
        return row * S + off

    def issue_ids(c_outer, m, k):
        _, ln = chunk_geom(m)
        p = tok_start(c_outer, m)
        pltpu.async_copy(ids_hbm.at[pl.ds(tok0 + p, ln)],
                         ids_cb.at[k, pl.ds(0, ln)], i_sems.at[k])
        pltpu.async_copy(tt_hbm.at[pl.ds(tok0 + p, ln)],
                         tt_cb.at[k, pl.ds(0, ln)], t_sems.at[k])

    def wait_ids(m, k):
        _, ln = chunk_geom(m)
        pltpu.make_async_copy(ids_hbm.at[pl.ds(tok0, ln)],
                              ids_cb.at[k, pl.ds(0, ln)], i_sems.at[k]).wait()

    def issue_gather(m, k):
        _, ln = chunk_geom(m)
        pltpu.async_copy(word_hbm.at[ids_cb.at[k, pl.ds(0, ln)]],
                         bufs[k].at[pl.ds(0, ln)], g_sems.at[k])

    for m in range(NBUF):
        issue_ids(0, m, m)
    for m in range(2):
        wait_ids(m, m)
        issue_gather(m, m)

    def outer_body(t, carry):
        for m in range(INNER):
            off, ln = chunk_geom(m)
            k = m % NBUF
            k2 = (m + 2) % NBUF
            _, ln2 = chunk_geom(m + 2)
            c = t * INNER + m
            row = t * (INNER // 2) + m // 2
            b = b0 + row
            p0 = row * S + off

            pltpu.make_async_copy(
                word_hbm.at[ids_cb.at[k, pl.ds(0, ln)]],
                bufs[k].at[pl.ds(0, ln)], g_sems.at[k]).wait()
            pltpu.make_async_copy(
                tt_hbm.at[pl.ds(tok0, ln)],
                tt_cb.at[k, pl.ds(0, ln)], t_sems.at[k]).wait()

            @pl.when(c >= 2)
            def _():
                pltpu.make_async_copy(
                    bufs[k2].at[pl.ds(0, ln2)],
                    out_hbm.at[b, pl.ds(0, ln2)], o_sems.at[k2]).wait()

            @pl.when(c + 2 < NCHUNK)
            def _():
                wait_ids(m + 2, k2)
                issue_gather(m + 2, k2)

            buf = bufs[k]

            g_regs = [g_v[pl.ds(j * L, L)] for j in range(NH)]
            b_regs = [b_v[pl.ds(j * L, L)] for j in range(NH)]

            @plsc.parallel_loop(0, ln, unroll=UNROLL)
            def tok_body(s):
                tt = tt_cb[k, pl.ds(s, L)][0]
                ptrow = tt * S + (off + s)
                xs = []
                qs = []
                for j in range(NH):
                    sl = pl.ds(j * L, L)
                    x = buf[s, sl] + pt_v[ptrow, sl]
                    xs.append(x)
                    qs.append(x * x)
                def tree(vs):
                    while len(vs) > 1:
                        vs = [a + b for a, b in zip(vs[::2], vs[1::2])]
                    return vs[0]
                acc_s = tree(xs)
                acc_q = tree(qs)
                acc_s = acc_s + _perm(acc_s, 8)
                acc_q = acc_q + _perm(acc_q, 8)
                z = jnp.where(jnp.arange(L) < 8, acc_s, acc_q)
                for d in (4, 2, 1):
                    z = z + _perm(z, d)
                mean = _bcast(z, 0) * (1.0 / H)
                ex2 = _bcast(z, 8) * (1.0 / H)
                inv = _rsqrt(ex2 - mean * mean + 1e-12)
                for j in range(NH):
                    sl = pl.ds(j * L, L)
                    buf[s, sl] = (xs[j] - mean) * inv * g_regs[j] + b_regs[j]

            pltpu.async_copy(
                buf.at[pl.ds(0, ln)],
                out_hbm.at[b, pl.ds(off, ln)], o_sems.at[k])

            @pl.when(c + NBUF < NCHUNK)
            def _():
                row4 = (c + NBUF) // 2
                off4, ln4 = chunk_geom(m)
                p4 = row4 * S + off4
                pltpu.async_copy(ids_hbm.at[pl.ds(tok0 + p4, ln4)],
                                 ids_cb.at[k, pl.ds(0, ln4)], i_sems.at[k])
                pltpu.async_copy(tt_hbm.at[pl.ds(tok0 + p4, ln4)],
                                 tt_cb.at[k, pl.ds(0, ln4)], t_sems.at[k])
        return carry

    lax.fori_loop(0, NCHUNK // INNER, outer_body, 0)

    for m in (INNER - 2, INNER - 1):
        off, ln = chunk_geom(m)
        k = m % NBUF
        pltpu.make_async_copy(
            bufs[k].at[pl.ds(0, ln)],
            out_hbm.at[b0, pl.ds(off, ln)], o_sems.at[k]).wait()


def kernel(input_ids, token_type_ids, word_embeddings, position_embeddings,
           token_type_embeddings, ln_gamma, ln_beta):
    mesh = plsc.VectorSubcoreMesh(core_axis_name="c", subcore_axis_name="s")
    f = pl.kernel(
        _sc_body,
        out_type=jax.ShapeDtypeStruct((B, S, H), jnp.float32),
        mesh=mesh,
        scratch_types=[
            pltpu.VMEM((C0, H), jnp.float32),
            pltpu.VMEM((C0, H), jnp.float32),
            pltpu.VMEM((C0, H), jnp.float32),
            pltpu.VMEM((C0, H), jnp.float32),
            pltpu.VMEM((NBUF, C0 + 8), jnp.int32),
            pltpu.VMEM((NBUF, C0 + L), jnp.int32),
            pltpu.VMEM((2 * S, H), jnp.float32),
            pltpu.VMEM((T, H), jnp.float32),
            pltpu.VMEM((H,), jnp.float32),
            pltpu.VMEM((H,), jnp.float32),
            pltpu.SemaphoreType.DMA((NBUF,)),
            pltpu.SemaphoreType.DMA((NBUF,)),
            pltpu.SemaphoreType.DMA((NBUF,)),
            pltpu.SemaphoreType.DMA((NBUF,)),
        ],
    )
    return f(input_ids.astype(jnp.int32).reshape(-1),
             token_type_ids.astype(jnp.int32).reshape(-1),
             word_embeddings, position_embeddings, token_type_embeddings,
             ln_gamma, ln_beta)
